# Initial kernel scaffold; baseline (speedup 1.0000x reference)
import jax
import jax.numpy as jnp
from jax.experimental import pallas as pl


def kernel(edge_feat, node_feat, edge_index):
    raise NotImplementedError("write your pallas kernel here")


# speedup vs baseline: 4.6356x; 4.6356x over previous
import functools

import jax
import jax.numpy as jnp
from jax import lax
from jax.experimental import pallas as pl
from jax.experimental.pallas import tpu as pltpu
from jax.experimental.pallas import tpu_sc as plsc

_L = 128
_K = 2
_C = _L * _K


def _build(E, D):
    NC, NS = 2, 16
    NW = NC * NS
    n_chunks = E // _C
    mesh = plsc.VectorSubcoreMesh(core_axis_name="c", subcore_axis_name="s")

    @functools.partial(
        pl.kernel,
        mesh=mesh,
        out_type=jax.ShapeDtypeStruct((E, 3 * D), jnp.float32),
        scratch_types=[
            pltpu.VMEM((_K, _L), jnp.int32),
            pltpu.VMEM((_K, _L), jnp.int32),
            pltpu.VMEM((_C, D), jnp.float32),
            pltpu.VMEM((_C, D), jnp.float32),
            pltpu.VMEM((_C, D), jnp.float32),
            pltpu.SemaphoreType.DMA,
        ],
    )
    def k(edge_hbm, node_hbm, sidx_hbm, didx_hbm, out_hbm,
          sidx_v, didx_v, erows_v, srows_v, drows_v, sem):
        wid = lax.axis_index("s") * NC + lax.axis_index("c")
        n_mine = (n_chunks - wid + NW - 1) // NW

        def body(i, carry):
            ch = wid + i * NW
            base = ch * _C
            pltpu.sync_copy(sidx_hbm.at[pl.ds(ch * _K, _K)], sidx_v)
            pltpu.sync_copy(didx_hbm.at[pl.ds(ch * _K, _K)], didx_v)
            copies = [pltpu.async_copy(
                edge_hbm.at[pl.ds(base, _C)], erows_v, sem)]
            for j in range(_K):
                copies.append(pltpu.async_copy(
                    node_hbm.at[sidx_v.at[j]],
                    srows_v.at[pl.ds(j * _L, _L)], sem))
                copies.append(pltpu.async_copy(
                    node_hbm.at[didx_v.at[j]],
                    drows_v.at[pl.ds(j * _L, _L)], sem))
            for c in copies:
                c.wait()
            pltpu.sync_copy(erows_v, out_hbm.at[pl.ds(base, _C), pl.ds(0, D)])
            pltpu.sync_copy(srows_v, out_hbm.at[pl.ds(base, _C), pl.ds(D, D)])
            pltpu.sync_copy(drows_v,
                            out_hbm.at[pl.ds(base, _C), pl.ds(2 * D, D)])
            return carry

        lax.fori_loop(0, n_mine, body, 0)

    return k


def kernel(edge_feat, node_feat, edge_index):
    E, D = edge_feat.shape
    idx = edge_index.astype(jnp.int32)
    sidx = idx[0].reshape(E // _L, _L)
    didx = idx[1].reshape(E // _L, _L)
    return _build(E, D)(edge_feat, node_feat, sidx, didx)


# speedup vs baseline: 5.3047x; 1.1444x over previous
import functools

import jax
import jax.numpy as jnp
from jax import lax
from jax.experimental import pallas as pl
from jax.experimental.pallas import tpu as pltpu
from jax.experimental.pallas import tpu_sc as plsc

_C = 128


def _build(E, D):
    NC, NS = 2, 16
    NW = NC * NS
    n_chunks = E // _C
    n_i = -(-n_chunks // NW)
    if n_i % 2:
        n_i += 1
    n_pairs = n_i // 2
    mesh = plsc.VectorSubcoreMesh(core_axis_name="c", subcore_axis_name="s")

    @functools.partial(
        pl.kernel,
        mesh=mesh,
        out_type=jax.ShapeDtypeStruct((E, 3 * D), jnp.float32),
        scratch_types=[
            pltpu.VMEM((2, _C), jnp.int32),
            pltpu.VMEM((2, _C), jnp.int32),
            pltpu.VMEM((_C, D), jnp.float32),
            pltpu.VMEM((_C, D), jnp.float32),
            pltpu.VMEM((_C, D), jnp.float32),
            pltpu.VMEM((_C, D), jnp.float32),
            pltpu.VMEM((_C, D), jnp.float32),
            pltpu.VMEM((_C, D), jnp.float32),
            pltpu.SemaphoreType.DMA,
            pltpu.SemaphoreType.DMA,
            pltpu.SemaphoreType.DMA,
            pltpu.SemaphoreType.DMA,
            pltpu.SemaphoreType.DMA,
        ],
    )
    def k(edge_hbm, node_hbm, idx_hbm, out_hbm,
          idx0, idx1, er0, sr0, dr0, er1, sr1, dr1,
          semidx, semg0, semg1, semst0, semst1):
        wid = lax.axis_index("s") * NC + lax.axis_index("c")
        idxb = (idx0, idx1)
        rowb = ((er0, sr0, dr0), (er1, sr1, dr1))
        semg = (semg0, semg1)
        semst = (semst0, semst1)

        def chunk_of(i):
            ch = wid + i * NW
            return jnp.where(ch >= n_chunks, ch - n_chunks, ch)

        def load_copies(i, b, sem):
            base = chunk_of(i) * _C
            er, sr, dr = rowb[b]
            return (
                pltpu.make_async_copy(edge_hbm.at[pl.ds(base, _C)], er, sem),
                pltpu.make_async_copy(node_hbm.at[idxb[b].at[0]], sr, sem),
                pltpu.make_async_copy(node_hbm.at[idxb[b].at[1]], dr, sem),
            )

        def store_copies(i, b, sem):
            base = chunk_of(i) * _C
            er, sr, dr = rowb[b]
            rows = pl.ds(base, _C)
            return (
                pltpu.make_async_copy(er, out_hbm.at[rows, pl.ds(0, D)], sem),
                pltpu.make_async_copy(sr, out_hbm.at[rows, pl.ds(D, D)], sem),
                pltpu.make_async_copy(dr, out_hbm.at[rows, pl.ds(2 * D, D)],
                                      sem),
            )

        def idx_copy(i, b):
            return pltpu.make_async_copy(idx_hbm.at[chunk_of(i)], idxb[b],
                                         semidx)

        def start(copies):
            for c in copies:
                c.start()

        def drain(copies):
            for c in copies:
                c.wait()

        idx_copy(0, 0).start()
        idx_copy(0, 0).wait()
        start(load_copies(0, 0, semg0))

        def body(p, carry):
            i0 = 2 * p
            i1 = i0 + 1

            idx_copy(i1, 1).start()
            drain(load_copies(i0, 0, semg0))
            start(store_copies(i0, 0, semst0))

            @pl.when(p > 0)
            def _():
                drain(store_copies(i0 - 1, 1, semst1))

            idx_copy(i1, 1).wait()
            start(load_copies(i1, 1, semg1))

            @pl.when(p < n_pairs - 1)
            def _():
                idx_copy(i1 + 1, 0).start()

            drain(load_copies(i1, 1, semg1))
            start(store_copies(i1, 1, semst1))
            drain(store_copies(i0, 0, semst0))

            @pl.when(p < n_pairs - 1)
            def _():
                idx_copy(i1 + 1, 0).wait()
                start(load_copies(i1 + 1, 0, semg0))

            return carry

        lax.fori_loop(0, n_pairs, body, 0)

        drain(store_copies(n_i - 1, 1, semst1))

    return k


def kernel(edge_feat, node_feat, edge_index):
    E, D = edge_feat.shape
    n_chunks = E // _C
    idx = edge_index.astype(jnp.int32)
    comb = idx.reshape(2, n_chunks, _C).transpose(1, 0, 2)
    return _build(E, D)(edge_feat, node_feat, comb)
